# Initial kernel scaffold; baseline (speedup 1.0000x reference)
import jax
import jax.numpy as jnp
from jax.experimental import pallas as pl


def kernel(x_lego, x_point, edge_index_ll, edge_index_pp, edge_index_lp, edge_index_pl, trans_Wq, trans_bq, trans_Wk, trans_bk, trans_Wv, trans_bv, trans_Ws, trans_bs, edge_W1, edge_b1, edge_W2, edge_b2, gatlp_W, gatlp_as, gatlp_ad, gatlp_b, gatpl_W, gatpl_as, gatpl_ad, gatpl_b):
    raise NotImplementedError("write your pallas kernel here")


# speedup vs baseline: 1.4808x; 1.4808x over previous
import functools

import jax
import jax.numpy as jnp
from jax import lax
from jax.experimental import pallas as pl
from jax.experimental.pallas import tpu as pltpu

D = 128
N = 10000
E = 320000

NB = 128
NBLK = (N + NB - 1) // NB
NPAD = NBLK * NB
EB = 5120
B = NBLK * EB
NEG = -1e30
SCALE = float(1.0 / (D ** 0.5))


def _prep_edges(ei):
    src, dst = ei[0], ei[1]
    dst_s, src_s = lax.sort((dst, src), num_keys=1)
    blk = dst_s // NB
    start = jnp.searchsorted(dst_s, jnp.arange(NBLK, dtype=jnp.int32) * NB)
    pos = jnp.arange(E, dtype=jnp.int32) - start[blk].astype(jnp.int32)
    slot = blk * EB + pos
    src_pad = jnp.zeros((B,), jnp.int32).at[slot].set(src_s, mode="drop")
    dstl_pad = jnp.full((B,), -1, jnp.int32).at[slot].set(
        dst_s - blk * NB, mode="drop")
    return (src_pad,
            dstl_pad.reshape(NBLK, 1, EB),
            dstl_pad.reshape(NBLK, EB, 1))


def _matmul_body(x_ref, w_ref, b_ref, o_ref):
    o_ref[...] = (
        jnp.dot(x_ref[...], w_ref[...], preferred_element_type=jnp.float32)
        + b_ref[...])


def _nmatmul(x, w, b):
    k = w.shape[1]
    rows = 1264
    return pl.pallas_call(
        _matmul_body,
        grid=(NPAD // rows,),
        in_specs=[
            pl.BlockSpec((rows, D), lambda i: (i, 0)),
            pl.BlockSpec((D, k), lambda i: (0, 0)),
            pl.BlockSpec((1, k), lambda i: (0, 0)),
        ],
        out_specs=pl.BlockSpec((rows, k), lambda i: (i, 0)),
        out_shape=jax.ShapeDtypeStruct((NPAD, k), jnp.float32),
    )(x, w, b.reshape(1, k))


def _gather_rows(table, idx):
    return jnp.take(table, idx, axis=0)


def _onehot(dstl_row):
    iota_n = lax.broadcasted_iota(jnp.int32, (NB, EB), 0)
    return (iota_n == dstl_row).astype(jnp.float32)


def _onehot_t(dstl_col):
    iota_n = lax.broadcasted_iota(jnp.int32, (EB, NB), 1)
    return (iota_n == dstl_col).astype(jnp.float32)


def _softmax_weights(alpha_col, dstl_row, dstl_col, m_t, m):
    valid_col = dstl_col >= 0
    w = jnp.where(m_t > 0.0, alpha_col, NEG)
    seg_max = jnp.max(w, axis=0, keepdims=True)
    seg_max_col = jnp.reshape(seg_max, (NB, 1))
    m_e = jnp.dot(m_t, seg_max_col,
                  preferred_element_type=jnp.float32)
    e_col = jnp.where(valid_col, jnp.exp(alpha_col - m_e), 0.0)
    s = jnp.dot(m, e_col, preferred_element_type=jnp.float32)
    return e_col, s


def _trans_body(dstl_r_ref, dstl_c_ref, q_ref, skip_ref, kg_ref, vg_ref,
                o_ref):
    dstl_row = dstl_r_ref[0]
    dstl_col = dstl_c_ref[...].reshape(EB, 1)
    m = _onehot(dstl_row)
    m_t = _onehot_t(dstl_col)
    q_exp = jnp.dot(m_t, q_ref[...], preferred_element_type=jnp.float32)
    alpha_col = jnp.sum(q_exp * kg_ref[...], axis=1, keepdims=True) * SCALE
    e_col, s = _softmax_weights(alpha_col, dstl_row, dstl_col, m_t, m)
    acc = jnp.dot(m, e_col * vg_ref[...], preferred_element_type=jnp.float32)
    o_ref[...] = acc / (s + 1e-16) + skip_ref[...]


def _trans_conv(x, prep, wq, bq, wk, bk, wv, bv, ws, bs):
    src_pad, dstl_r, dstl_c = prep
    wcat = jnp.concatenate([wq, wk, wv, ws], axis=1)
    bcat = jnp.concatenate([bq, bk, bv, bs], axis=0)
    qkvs = _nmatmul(x, wcat, bcat)
    kg = _gather_rows(qkvs[:, D:2 * D], src_pad)
    vg = _gather_rows(qkvs[:, 2 * D:3 * D], src_pad)
    return pl.pallas_call(
        _trans_body,
        grid=(NBLK,),
        in_specs=[
            pl.BlockSpec((1, 1, EB), lambda i: (i, 0, 0)),
            pl.BlockSpec((1, EB, 1), lambda i: (i, 0, 0)),
            pl.BlockSpec((NB, D), lambda i: (i, 0)),
            pl.BlockSpec((NB, D), lambda i: (i, 0)),
            pl.BlockSpec((EB, D), lambda i: (i, 0)),
            pl.BlockSpec((EB, D), lambda i: (i, 0)),
        ],
        out_specs=pl.BlockSpec((NB, D), lambda i: (i, 0)),
        out_shape=jax.ShapeDtypeStruct((NPAD, D), jnp.float32),
    )(dstl_r, dstl_c, qkvs[:, :D], qkvs[:, 3 * D:], kg, vg)


def _gat_body(dstl_r_ref, dstl_c_ref, hd_ref, base_ref, hsg_ref,
              as_ref, ad_ref, b_ref, o_ref):
    dstl_row = dstl_r_ref[0]
    dstl_col = dstl_c_ref[...].reshape(EB, 1)
    m = _onehot(dstl_row)
    m_t = _onehot_t(dstl_col)
    hsg = hsg_ref[...]
    s_src = jnp.sum(hsg * as_ref[...], axis=1, keepdims=True)
    s_dst = jnp.sum(hd_ref[...] * ad_ref[...], axis=1,
                    keepdims=True)
    logits = s_src + jnp.dot(m_t, s_dst,
                             preferred_element_type=jnp.float32)
    logits = jnp.where(logits > 0.0, logits, 0.2 * logits)
    e_col, s = _softmax_weights(logits, dstl_row, dstl_col, m_t, m)
    acc = jnp.dot(m, e_col * hsg, preferred_element_type=jnp.float32)
    o_ref[...] = acc / (s + 1e-16) + b_ref[...] + base_ref[...]


def _gat_conv(x_src, x_dst, prep, w, a_s, a_d, b, base):
    src_pad, dstl_r, dstl_c = prep
    zero = jnp.zeros((D,), jnp.float32)
    hs = _nmatmul(x_src, w, zero)
    hd = _nmatmul(x_dst, w, zero)
    hsg = _gather_rows(hs, src_pad)
    return pl.pallas_call(
        _gat_body,
        grid=(NBLK,),
        in_specs=[
            pl.BlockSpec((1, 1, EB), lambda i: (i, 0, 0)),
            pl.BlockSpec((1, EB, 1), lambda i: (i, 0, 0)),
            pl.BlockSpec((NB, D), lambda i: (i, 0)),
            pl.BlockSpec((NB, D), lambda i: (i, 0)),
            pl.BlockSpec((EB, D), lambda i: (i, 0)),
            pl.BlockSpec((1, D), lambda i: (0, 0)),
            pl.BlockSpec((1, D), lambda i: (0, 0)),
            pl.BlockSpec((1, D), lambda i: (0, 0)),
        ],
        out_specs=pl.BlockSpec((NB, D), lambda i: (i, 0)),
        out_shape=jax.ShapeDtypeStruct((NPAD, D), jnp.float32),
    )(dstl_r, dstl_c, hd, base, hsg, a_s.reshape(1, D), a_d.reshape(1, D),
      b.reshape(1, D))


_ECHUNK = 1280


def _edge_body(dstl_r_ref, dstl_c_ref, a_ref, xg_ref, w1b_ref, w2_ref,
               b2_ref, o_ref):
    dstl_row = dstl_r_ref[0]
    dstl_col = dstl_c_ref[...].reshape(EB, 1)
    valid_col = dstl_col >= 0
    m = _onehot(dstl_row)
    m_t = _onehot_t(dstl_col)

    h2_parts = []
    for c in range(EB // _ECHUNK):
        lo, hi = c * _ECHUNK, (c + 1) * _ECHUNK
        pre = (jnp.dot(m_t[lo:hi], a_ref[...],
                       preferred_element_type=jnp.float32)
               + jnp.dot(xg_ref[lo:hi, :], w1b_ref[...],
                         preferred_element_type=jnp.float32))
        h = jnp.maximum(pre, 0.0)
        h2_parts.append(
            jnp.dot(h, w2_ref[...], preferred_element_type=jnp.float32))
    h2 = jnp.concatenate(h2_parts, axis=0) + b2_ref[...]

    prev = jnp.concatenate(
        [jnp.full((1, 1), -2, jnp.int32), dstl_col[:-1]], axis=0)
    nxt = jnp.concatenate(
        [dstl_col[1:], jnp.full((1, 1), -2, jnp.int32)], axis=0)
    head = jnp.logical_or(dstl_col != prev, jnp.logical_not(valid_col))
    end = jnp.logical_and(valid_col, dstl_col != nxt)
    v = jnp.where(valid_col, h2, NEG)
    f = head.astype(jnp.float32)
    for d in (1, 2, 4, 8, 16, 32, 64):
        v_sh = jnp.concatenate(
            [jnp.full((d, D), NEG, jnp.float32), v[:-d]], axis=0)
        f_sh = jnp.concatenate(
            [jnp.ones((d, 1), jnp.float32), f[:-d]], axis=0)
        v = jnp.where(f > 0.0, v, jnp.maximum(v, v_sh))
        f = jnp.maximum(f, f_sh)

    picked = jnp.where(end, v, 0.0)
    o_ref[...] = jnp.dot(m, picked, preferred_element_type=jnp.float32)


def _edge_conv(x, prep, w1, b1, w2, b2):
    src_pad, dstl_r, dstl_c = prep
    w1_top, w1_bot = w1[:D], w1[D:]
    a = _nmatmul(x, w1_top - w1_bot, b1)
    xg = _gather_rows(x, src_pad)
    return pl.pallas_call(
        _edge_body,
        grid=(NBLK,),
        in_specs=[
            pl.BlockSpec((1, 1, EB), lambda i: (i, 0, 0)),
            pl.BlockSpec((1, EB, 1), lambda i: (i, 0, 0)),
            pl.BlockSpec((NB, 4 * D), lambda i: (i, 0)),
            pl.BlockSpec((EB, D), lambda i: (i, 0)),
            pl.BlockSpec((D, 4 * D), lambda i: (0, 0)),
            pl.BlockSpec((4 * D, D), lambda i: (0, 0)),
            pl.BlockSpec((1, D), lambda i: (0, 0)),
        ],
        out_specs=pl.BlockSpec((NB, D), lambda i: (i, 0)),
        out_shape=jax.ShapeDtypeStruct((NPAD, D), jnp.float32),
    )(dstl_r, dstl_c, a, xg, w1_bot, w2, b2.reshape(1, D))


def kernel(x_lego, x_point, edge_index_ll, edge_index_pp, edge_index_lp,
           edge_index_pl, trans_Wq, trans_bq, trans_Wk, trans_bk, trans_Wv,
           trans_bv, trans_Ws, trans_bs, edge_W1, edge_b1, edge_W2, edge_b2,
           gatlp_W, gatlp_as, gatlp_ad, gatlp_b, gatpl_W, gatpl_as,
           gatpl_ad, gatpl_b):
    pad = ((0, NPAD - N), (0, 0))
    lego = jnp.pad(x_lego, pad)
    point = jnp.pad(x_point, pad)

    p_ll = _prep_edges(edge_index_ll)
    p_pp = _prep_edges(edge_index_pp)
    p_lp = _prep_edges(edge_index_lp)
    p_pl = _prep_edges(edge_index_pl)

    for l in range(2):
        sa, sb = 2 * l, 2 * l + 1
        lg = _trans_conv(lego, p_ll, trans_Wq[sa], trans_bq[sa],
                         trans_Wk[sa], trans_bk[sa], trans_Wv[sa],
                         trans_bv[sa], trans_Ws[sa], trans_bs[sa])
        lg = _gat_conv(point, lego, p_pl, gatpl_W[l], gatpl_as[l],
                       gatpl_ad[l], gatpl_b[l], lg)
        pt = _edge_conv(point, p_pp, edge_W1[sa], edge_b1[sa],
                        edge_W2[sa], edge_b2[sa])
        pt = _gat_conv(lego, point, p_lp, gatlp_W[l], gatlp_as[l],
                       gatlp_ad[l], gatlp_b[l], pt)
        lego = _trans_conv(lg, p_ll, trans_Wq[sb], trans_bq[sb],
                           trans_Wk[sb], trans_bk[sb], trans_Wv[sb],
                           trans_bv[sb], trans_Ws[sb], trans_bs[sb])
        point = _edge_conv(pt, p_pp, edge_W1[sb], edge_b1[sb],
                           edge_W2[sb], edge_b2[sb])

    return lego[:N], point[:N]


# speedup vs baseline: 2.6690x; 1.8023x over previous
import functools

import jax
import jax.numpy as jnp
from jax import lax
from jax.experimental import pallas as pl
from jax.experimental.pallas import tpu as pltpu
from jax.experimental.pallas import tpu_sc as plsc

D = 128
N = 10000
E = 320000

NB = 128
NBLK = (N + NB - 1) // NB
NPAD = NBLK * NB
EB = 5120
B = NBLK * EB
NEG = -1e30
SCALE = float(1.0 / (D ** 0.5))
_PREC = lax.Precision.HIGHEST


def _prep_edges(ei):
    src, dst = ei[0], ei[1]
    dst_s, src_s = lax.sort((dst, src), num_keys=1)
    blk = dst_s // NB
    start = jnp.searchsorted(dst_s, jnp.arange(NBLK, dtype=jnp.int32) * NB)
    pos = jnp.arange(E, dtype=jnp.int32) - start[blk].astype(jnp.int32)
    slot = blk * EB + pos
    src_pad = jnp.zeros((B,), jnp.int32).at[slot].set(src_s, mode="drop")
    dstl_pad = jnp.full((B,), -1, jnp.int32).at[slot].set(
        dst_s - blk * NB, mode="drop")
    return (src_pad,
            dstl_pad.reshape(NBLK, 1, EB),
            dstl_pad.reshape(NBLK, EB, 1))


def _matmul_body(x_ref, w_ref, b_ref, o_ref):
    o_ref[...] = (
        jnp.dot(x_ref[...], w_ref[...], preferred_element_type=jnp.float32, precision=_PREC)
        + b_ref[...])


def _nmatmul(x, w, b):
    k = w.shape[1]
    rows = 1264
    return pl.pallas_call(
        _matmul_body,
        grid=(NPAD // rows,),
        in_specs=[
            pl.BlockSpec((rows, D), lambda i: (i, 0)),
            pl.BlockSpec((D, k), lambda i: (0, 0)),
            pl.BlockSpec((1, k), lambda i: (0, 0)),
        ],
        out_specs=pl.BlockSpec((rows, k), lambda i: (i, 0)),
        out_shape=jax.ShapeDtypeStruct((NPAD, k), jnp.float32),
    )(x, w, b.reshape(1, k))


_SC_MESH = plsc.VectorSubcoreMesh(core_axis_name="c", subcore_axis_name="s")
_GW = 128


def _gather_rows(table, idx):
    width = table.shape[1]

    @functools.partial(
        pl.kernel,
        out_type=jax.ShapeDtypeStruct((B, width), jnp.float32),
        mesh=_SC_MESH)
    def gk(tab_hbm, idx_hbm, out_hbm):
        def body(i_vmem, o_vmem):
            pltpu.sync_copy(tab_hbm.at[i_vmem.at[0]], o_vmem)

        pltpu.emit_pipeline(
            body,
            grid=(B // _GW,),
            in_specs=[pl.BlockSpec((1, _GW), lambda i: (0, i))],
            out_specs=[pl.BlockSpec((_GW, width), lambda i: (i, 0))],
            core_axis_name=("c", "s"),
            dimension_semantics=(pltpu.PARALLEL,),
        )(idx_hbm, out_hbm)

    return gk(table, idx.reshape(1, B))


def _onehot(dstl_row):
    iota_n = lax.broadcasted_iota(jnp.int32, (NB, EB), 0)
    return (iota_n == dstl_row).astype(jnp.float32)


def _onehot_t(dstl_col):
    iota_n = lax.broadcasted_iota(jnp.int32, (EB, NB), 1)
    return (iota_n == dstl_col).astype(jnp.float32)


def _softmax_weights(alpha_col, dstl_row, dstl_col, m_t, m):
    valid_col = dstl_col >= 0
    w = jnp.where(m_t > 0.0, alpha_col, NEG)
    seg_max = jnp.max(w, axis=0, keepdims=True)
    seg_max_col = jnp.reshape(seg_max, (NB, 1))
    m_e = jnp.dot(m_t, seg_max_col,
                  preferred_element_type=jnp.float32, precision=_PREC)
    e_col = jnp.where(valid_col, jnp.exp(alpha_col - m_e), 0.0)
    s = jnp.dot(m, e_col, preferred_element_type=jnp.float32, precision=_PREC)
    return e_col, s


def _trans_body(dstl_r_ref, dstl_c_ref, q_ref, skip_ref, kvg_ref, o_ref):
    dstl_row = dstl_r_ref[0]
    dstl_col = dstl_c_ref[...].reshape(EB, 1)
    m = _onehot(dstl_row)
    m_t = _onehot_t(dstl_col)
    kg = kvg_ref[:, :D]
    vg = kvg_ref[:, D:]
    q_exp = jnp.dot(m_t, q_ref[...], preferred_element_type=jnp.float32, precision=_PREC)
    alpha_col = jnp.sum(q_exp * kg, axis=1, keepdims=True) * SCALE
    e_col, s = _softmax_weights(alpha_col, dstl_row, dstl_col, m_t, m)
    acc = jnp.dot(m, e_col * vg, preferred_element_type=jnp.float32, precision=_PREC)
    o_ref[...] = acc / (s + 1e-16) + skip_ref[...]


def _trans_conv(x, prep, wq, bq, wk, bk, wv, bv, ws, bs):
    src_pad, dstl_r, dstl_c = prep
    wcat = jnp.concatenate([wq, wk, wv, ws], axis=1)
    bcat = jnp.concatenate([bq, bk, bv, bs], axis=0)
    qkvs = _nmatmul(x, wcat, bcat)
    kvg = _gather_rows(qkvs[:, D:3 * D], src_pad)
    return pl.pallas_call(
        _trans_body,
        grid=(NBLK,),
        in_specs=[
            pl.BlockSpec((1, 1, EB), lambda i: (i, 0, 0)),
            pl.BlockSpec((1, EB, 1), lambda i: (i, 0, 0)),
            pl.BlockSpec((NB, D), lambda i: (i, 0)),
            pl.BlockSpec((NB, D), lambda i: (i, 0)),
            pl.BlockSpec((EB, 2 * D), lambda i: (i, 0)),
        ],
        out_specs=pl.BlockSpec((NB, D), lambda i: (i, 0)),
        out_shape=jax.ShapeDtypeStruct((NPAD, D), jnp.float32),
    )(dstl_r, dstl_c, qkvs[:, :D], qkvs[:, 3 * D:], kvg)


def _gat_body(dstl_r_ref, dstl_c_ref, hd_ref, base_ref, hsg_ref,
              as_ref, ad_ref, b_ref, o_ref):
    dstl_row = dstl_r_ref[0]
    dstl_col = dstl_c_ref[...].reshape(EB, 1)
    m = _onehot(dstl_row)
    m_t = _onehot_t(dstl_col)
    hsg = hsg_ref[...]
    s_src = jnp.sum(hsg * as_ref[...], axis=1, keepdims=True)
    s_dst = jnp.sum(hd_ref[...] * ad_ref[...], axis=1,
                    keepdims=True)
    logits = s_src + jnp.dot(m_t, s_dst,
                             preferred_element_type=jnp.float32, precision=_PREC)
    logits = jnp.where(logits > 0.0, logits, 0.2 * logits)
    e_col, s = _softmax_weights(logits, dstl_row, dstl_col, m_t, m)
    acc = jnp.dot(m, e_col * hsg, preferred_element_type=jnp.float32, precision=_PREC)
    o_ref[...] = acc / (s + 1e-16) + b_ref[...] + base_ref[...]


def _gat_conv(x_src, x_dst, prep, w, a_s, a_d, b, base):
    src_pad, dstl_r, dstl_c = prep
    zero = jnp.zeros((D,), jnp.float32)
    hs = _nmatmul(x_src, w, zero)
    hd = _nmatmul(x_dst, w, zero)
    hsg = _gather_rows(hs, src_pad)
    return pl.pallas_call(
        _gat_body,
        grid=(NBLK,),
        in_specs=[
            pl.BlockSpec((1, 1, EB), lambda i: (i, 0, 0)),
            pl.BlockSpec((1, EB, 1), lambda i: (i, 0, 0)),
            pl.BlockSpec((NB, D), lambda i: (i, 0)),
            pl.BlockSpec((NB, D), lambda i: (i, 0)),
            pl.BlockSpec((EB, D), lambda i: (i, 0)),
            pl.BlockSpec((1, D), lambda i: (0, 0)),
            pl.BlockSpec((1, D), lambda i: (0, 0)),
            pl.BlockSpec((1, D), lambda i: (0, 0)),
        ],
        out_specs=pl.BlockSpec((NB, D), lambda i: (i, 0)),
        out_shape=jax.ShapeDtypeStruct((NPAD, D), jnp.float32),
    )(dstl_r, dstl_c, hd, base, hsg, a_s.reshape(1, D), a_d.reshape(1, D),
      b.reshape(1, D))


_ECHUNK = 1280


def _edge_body(dstl_r_ref, dstl_c_ref, a_ref, xg_ref, w1b_ref, w2_ref,
               b2_ref, o_ref):
    dstl_row = dstl_r_ref[0]
    dstl_col = dstl_c_ref[...].reshape(EB, 1)
    valid_col = dstl_col >= 0
    m = _onehot(dstl_row)
    m_t = _onehot_t(dstl_col)

    h2_parts = []
    for c in range(EB // _ECHUNK):
        lo, hi = c * _ECHUNK, (c + 1) * _ECHUNK
        pre = (jnp.dot(m_t[lo:hi], a_ref[...],
                       preferred_element_type=jnp.float32, precision=_PREC)
               + jnp.dot(xg_ref[lo:hi, :], w1b_ref[...],
                         preferred_element_type=jnp.float32, precision=_PREC))
        h = jnp.maximum(pre, 0.0)
        h2_parts.append(
            jnp.dot(h, w2_ref[...], preferred_element_type=jnp.float32, precision=_PREC))
    h2 = jnp.concatenate(h2_parts, axis=0) + b2_ref[...]

    prev = jnp.concatenate(
        [jnp.full((1, 1), -2, jnp.int32), dstl_col[:-1]], axis=0)
    nxt = jnp.concatenate(
        [dstl_col[1:], jnp.full((1, 1), -2, jnp.int32)], axis=0)
    head = jnp.logical_or(dstl_col != prev, jnp.logical_not(valid_col))
    end = jnp.logical_and(valid_col, dstl_col != nxt)
    v = jnp.where(valid_col, h2, NEG)
    f = head.astype(jnp.float32)
    for d in (1, 2, 4, 8, 16, 32, 64):
        v_sh = jnp.concatenate(
            [jnp.full((d, D), NEG, jnp.float32), v[:-d]], axis=0)
        f_sh = jnp.concatenate(
            [jnp.ones((d, 1), jnp.float32), f[:-d]], axis=0)
        v = jnp.where(f > 0.0, v, jnp.maximum(v, v_sh))
        f = jnp.maximum(f, f_sh)

    picked = jnp.where(end, v, 0.0)
    o_ref[...] = jnp.dot(m, picked, preferred_element_type=jnp.float32, precision=_PREC)


def _edge_conv(x, prep, w1, b1, w2, b2):
    src_pad, dstl_r, dstl_c = prep
    w1_top, w1_bot = w1[:D], w1[D:]
    a = _nmatmul(x, w1_top - w1_bot, b1)
    xg = _gather_rows(x, src_pad)
    return pl.pallas_call(
        _edge_body,
        grid=(NBLK,),
        in_specs=[
            pl.BlockSpec((1, 1, EB), lambda i: (i, 0, 0)),
            pl.BlockSpec((1, EB, 1), lambda i: (i, 0, 0)),
            pl.BlockSpec((NB, 4 * D), lambda i: (i, 0)),
            pl.BlockSpec((EB, D), lambda i: (i, 0)),
            pl.BlockSpec((D, 4 * D), lambda i: (0, 0)),
            pl.BlockSpec((4 * D, D), lambda i: (0, 0)),
            pl.BlockSpec((1, D), lambda i: (0, 0)),
        ],
        out_specs=pl.BlockSpec((NB, D), lambda i: (i, 0)),
        out_shape=jax.ShapeDtypeStruct((NPAD, D), jnp.float32),
    )(dstl_r, dstl_c, a, xg, w1_bot, w2, b2.reshape(1, D))


def kernel(x_lego, x_point, edge_index_ll, edge_index_pp, edge_index_lp,
           edge_index_pl, trans_Wq, trans_bq, trans_Wk, trans_bk, trans_Wv,
           trans_bv, trans_Ws, trans_bs, edge_W1, edge_b1, edge_W2, edge_b2,
           gatlp_W, gatlp_as, gatlp_ad, gatlp_b, gatpl_W, gatpl_as,
           gatpl_ad, gatpl_b):
    pad = ((0, NPAD - N), (0, 0))
    lego = jnp.pad(x_lego, pad)
    point = jnp.pad(x_point, pad)

    p_ll = _prep_edges(edge_index_ll)
    p_pp = _prep_edges(edge_index_pp)
    p_lp = _prep_edges(edge_index_lp)
    p_pl = _prep_edges(edge_index_pl)

    for l in range(2):
        sa, sb = 2 * l, 2 * l + 1
        lg = _trans_conv(lego, p_ll, trans_Wq[sa], trans_bq[sa],
                         trans_Wk[sa], trans_bk[sa], trans_Wv[sa],
                         trans_bv[sa], trans_Ws[sa], trans_bs[sa])
        lg = _gat_conv(point, lego, p_pl, gatpl_W[l], gatpl_as[l],
                       gatpl_ad[l], gatpl_b[l], lg)
        pt = _edge_conv(point, p_pp, edge_W1[sa], edge_b1[sa],
                        edge_W2[sa], edge_b2[sa])
        pt = _gat_conv(lego, point, p_lp, gatlp_W[l], gatlp_as[l],
                       gatlp_ad[l], gatlp_b[l], pt)
        lego = _trans_conv(lg, p_ll, trans_Wq[sb], trans_bq[sb],
                           trans_Wk[sb], trans_bk[sb], trans_Wv[sb],
                           trans_bv[sb], trans_Ws[sb], trans_bs[sb])
        point = _edge_conv(pt, p_pp, edge_W1[sb], edge_b1[sb],
                           edge_W2[sb], edge_b2[sb])

    return lego[:N], point[:N]


# speedup vs baseline: 2.6693x; 1.0001x over previous
import functools

import jax
import jax.numpy as jnp
from jax import lax
from jax.experimental import pallas as pl
from jax.experimental.pallas import tpu as pltpu
from jax.experimental.pallas import tpu_sc as plsc

D = 128
N = 10000
E = 320000

NB = 128
NBLK = (N + NB - 1) // NB
NPAD = NBLK * NB
EB = 5120
B = NBLK * EB
NEG = -1e30
SCALE = float(1.0 / (D ** 0.5))
_PREC = lax.Precision.HIGHEST


def _prep_edges(ei):
    src, dst = ei[0], ei[1]
    key = lax.sort(dst * 16384 + src)
    dst_s = key >> 14
    src_s = key & 16383
    blk = dst_s // NB
    start = jnp.searchsorted(dst_s, jnp.arange(NBLK, dtype=jnp.int32) * NB)
    pos = jnp.arange(E, dtype=jnp.int32) - start[blk].astype(jnp.int32)
    slot = blk * EB + pos
    src_pad = jnp.zeros((B,), jnp.int32).at[slot].set(src_s, mode="drop")
    dstl_pad = jnp.full((B,), -1, jnp.int32).at[slot].set(
        dst_s - blk * NB, mode="drop")
    return (src_pad,
            dstl_pad.reshape(NBLK, 1, EB),
            dstl_pad.reshape(NBLK, EB, 1))


def _matmul_body(x_ref, w_ref, b_ref, o_ref):
    o_ref[...] = (
        jnp.dot(x_ref[...], w_ref[...], preferred_element_type=jnp.float32, precision=_PREC)
        + b_ref[...])


def _nmatmul(x, w, b):
    k = w.shape[1]
    rows = 1264
    return pl.pallas_call(
        _matmul_body,
        grid=(NPAD // rows,),
        in_specs=[
            pl.BlockSpec((rows, D), lambda i: (i, 0)),
            pl.BlockSpec((D, k), lambda i: (0, 0)),
            pl.BlockSpec((1, k), lambda i: (0, 0)),
        ],
        out_specs=pl.BlockSpec((rows, k), lambda i: (i, 0)),
        out_shape=jax.ShapeDtypeStruct((NPAD, k), jnp.float32),
    )(x, w, b.reshape(1, k))


_SC_MESH = plsc.VectorSubcoreMesh(core_axis_name="c", subcore_axis_name="s")
_GW = 128


def _gather_rows(table, idx):
    width = table.shape[1]

    @functools.partial(
        pl.kernel,
        out_type=jax.ShapeDtypeStruct((B, width), jnp.float32),
        mesh=_SC_MESH)
    def gk(tab_hbm, idx_hbm, out_hbm):
        def body(i_vmem, o_vmem):
            pltpu.sync_copy(tab_hbm.at[i_vmem.at[0]], o_vmem)

        pltpu.emit_pipeline(
            body,
            grid=(B // _GW,),
            in_specs=[pl.BlockSpec((1, _GW), lambda i: (0, i))],
            out_specs=[pl.BlockSpec((_GW, width), lambda i: (i, 0))],
            core_axis_name=("c", "s"),
            dimension_semantics=(pltpu.PARALLEL,),
        )(idx_hbm, out_hbm)

    return gk(table, idx.reshape(1, B))


def _onehot(dstl_row):
    iota_n = lax.broadcasted_iota(jnp.int32, (NB, EB), 0)
    return (iota_n == dstl_row).astype(jnp.float32)


def _onehot_t(dstl_col):
    iota_n = lax.broadcasted_iota(jnp.int32, (EB, NB), 1)
    return (iota_n == dstl_col).astype(jnp.float32)


def _softmax_weights(alpha_col, dstl_row, dstl_col, m_t, m):
    valid_col = dstl_col >= 0
    w = jnp.where(m_t > 0.0, alpha_col, NEG)
    seg_max = jnp.max(w, axis=0, keepdims=True)
    seg_max_col = jnp.reshape(seg_max, (NB, 1))
    m_e = jnp.dot(m_t, seg_max_col,
                  preferred_element_type=jnp.float32, precision=_PREC)
    e_col = jnp.where(valid_col, jnp.exp(alpha_col - m_e), 0.0)
    s = jnp.dot(m, e_col, preferred_element_type=jnp.float32, precision=_PREC)
    return e_col, s


def _trans_body(dstl_r_ref, dstl_c_ref, q_ref, skip_ref, kvg_ref, o_ref):
    dstl_row = dstl_r_ref[0]
    dstl_col = dstl_c_ref[...].reshape(EB, 1)
    m = _onehot(dstl_row)
    m_t = _onehot_t(dstl_col)
    kg = kvg_ref[:, :D]
    vg = kvg_ref[:, D:]
    q_exp = jnp.dot(m_t, q_ref[...], preferred_element_type=jnp.float32, precision=_PREC)
    alpha_col = jnp.sum(q_exp * kg, axis=1, keepdims=True) * SCALE
    e_col, s = _softmax_weights(alpha_col, dstl_row, dstl_col, m_t, m)
    acc = jnp.dot(m, e_col * vg, preferred_element_type=jnp.float32, precision=_PREC)
    o_ref[...] = acc / (s + 1e-16) + skip_ref[...]


def _trans_conv(x, prep, wq, bq, wk, bk, wv, bv, ws, bs):
    src_pad, dstl_r, dstl_c = prep
    wcat = jnp.concatenate([wq, wk, wv, ws], axis=1)
    bcat = jnp.concatenate([bq, bk, bv, bs], axis=0)
    qkvs = _nmatmul(x, wcat, bcat)
    kvg = _gather_rows(qkvs[:, D:3 * D], src_pad)
    return pl.pallas_call(
        _trans_body,
        grid=(NBLK,),
        in_specs=[
            pl.BlockSpec((1, 1, EB), lambda i: (i, 0, 0)),
            pl.BlockSpec((1, EB, 1), lambda i: (i, 0, 0)),
            pl.BlockSpec((NB, D), lambda i: (i, 0)),
            pl.BlockSpec((NB, D), lambda i: (i, 0)),
            pl.BlockSpec((EB, 2 * D), lambda i: (i, 0)),
        ],
        out_specs=pl.BlockSpec((NB, D), lambda i: (i, 0)),
        out_shape=jax.ShapeDtypeStruct((NPAD, D), jnp.float32),
    )(dstl_r, dstl_c, qkvs[:, :D], qkvs[:, 3 * D:], kvg)


def _gat_body(dstl_r_ref, dstl_c_ref, hd_ref, base_ref, hsg_ref,
              as_ref, ad_ref, b_ref, o_ref):
    dstl_row = dstl_r_ref[0]
    dstl_col = dstl_c_ref[...].reshape(EB, 1)
    m = _onehot(dstl_row)
    m_t = _onehot_t(dstl_col)
    hsg = hsg_ref[...]
    s_src = jnp.sum(hsg * as_ref[...], axis=1, keepdims=True)
    s_dst = jnp.sum(hd_ref[...] * ad_ref[...], axis=1,
                    keepdims=True)
    logits = s_src + jnp.dot(m_t, s_dst,
                             preferred_element_type=jnp.float32, precision=_PREC)
    logits = jnp.where(logits > 0.0, logits, 0.2 * logits)
    e_col, s = _softmax_weights(logits, dstl_row, dstl_col, m_t, m)
    acc = jnp.dot(m, e_col * hsg, preferred_element_type=jnp.float32, precision=_PREC)
    o_ref[...] = acc / (s + 1e-16) + b_ref[...] + base_ref[...]


def _gat_conv(x_src, x_dst, prep, w, a_s, a_d, b, base):
    src_pad, dstl_r, dstl_c = prep
    zero = jnp.zeros((D,), jnp.float32)
    hs = _nmatmul(x_src, w, zero)
    hd = _nmatmul(x_dst, w, zero)
    hsg = _gather_rows(hs, src_pad)
    return pl.pallas_call(
        _gat_body,
        grid=(NBLK,),
        in_specs=[
            pl.BlockSpec((1, 1, EB), lambda i: (i, 0, 0)),
            pl.BlockSpec((1, EB, 1), lambda i: (i, 0, 0)),
            pl.BlockSpec((NB, D), lambda i: (i, 0)),
            pl.BlockSpec((NB, D), lambda i: (i, 0)),
            pl.BlockSpec((EB, D), lambda i: (i, 0)),
            pl.BlockSpec((1, D), lambda i: (0, 0)),
            pl.BlockSpec((1, D), lambda i: (0, 0)),
            pl.BlockSpec((1, D), lambda i: (0, 0)),
        ],
        out_specs=pl.BlockSpec((NB, D), lambda i: (i, 0)),
        out_shape=jax.ShapeDtypeStruct((NPAD, D), jnp.float32),
    )(dstl_r, dstl_c, hd, base, hsg, a_s.reshape(1, D), a_d.reshape(1, D),
      b.reshape(1, D))


_ECHUNK = 1280


def _edge_body(dstl_r_ref, dstl_c_ref, a_ref, xg_ref, w1b_ref, w2_ref,
               b2_ref, o_ref):
    dstl_row = dstl_r_ref[0]
    dstl_col = dstl_c_ref[...].reshape(EB, 1)
    valid_col = dstl_col >= 0
    m = _onehot(dstl_row)
    m_t = _onehot_t(dstl_col)

    h2_parts = []
    for c in range(EB // _ECHUNK):
        lo, hi = c * _ECHUNK, (c + 1) * _ECHUNK
        pre = (jnp.dot(m_t[lo:hi], a_ref[...],
                       preferred_element_type=jnp.float32, precision=_PREC)
               + jnp.dot(xg_ref[lo:hi, :], w1b_ref[...],
                         preferred_element_type=jnp.float32, precision=_PREC))
        h = jnp.maximum(pre, 0.0)
        h2_parts.append(
            jnp.dot(h, w2_ref[...], preferred_element_type=jnp.float32, precision=_PREC))
    h2 = jnp.concatenate(h2_parts, axis=0) + b2_ref[...]

    prev = jnp.concatenate(
        [jnp.full((1, 1), -2, jnp.int32), dstl_col[:-1]], axis=0)
    nxt = jnp.concatenate(
        [dstl_col[1:], jnp.full((1, 1), -2, jnp.int32)], axis=0)
    head = jnp.logical_or(dstl_col != prev, jnp.logical_not(valid_col))
    end = jnp.logical_and(valid_col, dstl_col != nxt)
    v = jnp.where(valid_col, h2, NEG)
    f = head.astype(jnp.float32)
    for d in (1, 2, 4, 8, 16, 32, 64):
        v_sh = jnp.concatenate(
            [jnp.full((d, D), NEG, jnp.float32), v[:-d]], axis=0)
        f_sh = jnp.concatenate(
            [jnp.ones((d, 1), jnp.float32), f[:-d]], axis=0)
        v = jnp.where(f > 0.0, v, jnp.maximum(v, v_sh))
        f = jnp.maximum(f, f_sh)

    picked = jnp.where(end, v, 0.0)
    o_ref[...] = jnp.dot(m, picked, preferred_element_type=jnp.float32, precision=_PREC)


def _edge_conv(x, prep, w1, b1, w2, b2):
    src_pad, dstl_r, dstl_c = prep
    w1_top, w1_bot = w1[:D], w1[D:]
    a = _nmatmul(x, w1_top - w1_bot, b1)
    xg = _gather_rows(x, src_pad)
    return pl.pallas_call(
        _edge_body,
        grid=(NBLK,),
        in_specs=[
            pl.BlockSpec((1, 1, EB), lambda i: (i, 0, 0)),
            pl.BlockSpec((1, EB, 1), lambda i: (i, 0, 0)),
            pl.BlockSpec((NB, 4 * D), lambda i: (i, 0)),
            pl.BlockSpec((EB, D), lambda i: (i, 0)),
            pl.BlockSpec((D, 4 * D), lambda i: (0, 0)),
            pl.BlockSpec((4 * D, D), lambda i: (0, 0)),
            pl.BlockSpec((1, D), lambda i: (0, 0)),
        ],
        out_specs=pl.BlockSpec((NB, D), lambda i: (i, 0)),
        out_shape=jax.ShapeDtypeStruct((NPAD, D), jnp.float32),
    )(dstl_r, dstl_c, a, xg, w1_bot, w2, b2.reshape(1, D))


def kernel(x_lego, x_point, edge_index_ll, edge_index_pp, edge_index_lp,
           edge_index_pl, trans_Wq, trans_bq, trans_Wk, trans_bk, trans_Wv,
           trans_bv, trans_Ws, trans_bs, edge_W1, edge_b1, edge_W2, edge_b2,
           gatlp_W, gatlp_as, gatlp_ad, gatlp_b, gatpl_W, gatpl_as,
           gatpl_ad, gatpl_b):
    pad = ((0, NPAD - N), (0, 0))
    lego = jnp.pad(x_lego, pad)
    point = jnp.pad(x_point, pad)

    p_ll = _prep_edges(edge_index_ll)
    p_pp = _prep_edges(edge_index_pp)
    p_lp = _prep_edges(edge_index_lp)
    p_pl = _prep_edges(edge_index_pl)

    for l in range(2):
        sa, sb = 2 * l, 2 * l + 1
        lg = _trans_conv(lego, p_ll, trans_Wq[sa], trans_bq[sa],
                         trans_Wk[sa], trans_bk[sa], trans_Wv[sa],
                         trans_bv[sa], trans_Ws[sa], trans_bs[sa])
        lg = _gat_conv(point, lego, p_pl, gatpl_W[l], gatpl_as[l],
                       gatpl_ad[l], gatpl_b[l], lg)
        pt = _edge_conv(point, p_pp, edge_W1[sa], edge_b1[sa],
                        edge_W2[sa], edge_b2[sa])
        pt = _gat_conv(lego, point, p_lp, gatlp_W[l], gatlp_as[l],
                       gatlp_ad[l], gatlp_b[l], pt)
        lego = _trans_conv(lg, p_ll, trans_Wq[sb], trans_bq[sb],
                           trans_Wk[sb], trans_bk[sb], trans_Wv[sb],
                           trans_bv[sb], trans_Ws[sb], trans_bs[sb])
        point = _edge_conv(pt, p_pp, edge_W1[sb], edge_b1[sb],
                           edge_W2[sb], edge_b2[sb])

    return lego[:N], point[:N]


# speedup vs baseline: 6.2444x; 2.3394x over previous
import dataclasses
import functools

import jax
import jax.numpy as jnp
from jax import lax
from jax.experimental import pallas as pl
from jax.experimental.pallas import tpu as pltpu
from jax.experimental.pallas import tpu_sc as plsc

D = 128
N = 10000
E = 320000

NB = 128
NBLK = (N + NB - 1) // NB
NPAD = NBLK * NB
EB = 5120
B = NBLK * EB
NEG = -1e30
SCALE = float(1.0 / (D ** 0.5))
_PREC = lax.Precision.HIGHEST


_NSUB = 16
_EPW = E // _NSUB
_NB_FULL = -(-_EPW // 128)
_CHUNK = 2048
_EPAD = E + _CHUNK
_HD = 10112
_DUMP = 10111


def _permute16(x, idx):
    dn = lax.GatherDimensionNumbers(
        offset_dims=(), collapsed_slice_dims=(0,), start_index_map=(0,))
    return lax.gather(x, idx[:, None], dn, (1,),
                      mode=lax.GatherScatterMode.PROMISE_IN_BOUNDS)


def _rank_and_last(d):
    io = jnp.arange(16, dtype=jnp.int32)

    def step(sh, carry):
        rank, after = carry
        prv = _permute16(d, jnp.maximum(io - sh, 0))
        nxt = _permute16(d, jnp.minimum(io + sh, 15))
        rank = rank + jnp.where((prv == d) & (io >= sh), 1, 0)
        after = after + jnp.where((nxt == d) & (io + sh < 16), 1, 0)
        return rank, after

    zero = jnp.zeros((16,), jnp.int32)
    rank, after = lax.fori_loop(1, 16, step, (zero, zero))
    return rank, after == 0


def _pack_body(ei_hbm, src_out, dstl_out, cnt_out, hist, prev, tot, run,
               row, dstage, sstage, sidx, sdat, ddat, shared):
    core = lax.axis_index("c")
    wid = lax.axis_index("s")
    io = jnp.arange(16, dtype=jnp.int32)
    ones = jnp.ones((16,), jnp.int32)

    for k in range(2):
        s_id = 2 * core + k

        @pl.loop(0, _HD // 16)
        def _(i):
            hist[pl.ds(i * 16, 16)] = jnp.zeros((16,), jnp.int32)

        def edge_groups(b, use_src, fn):
            @pl.when(b % 16 == 0)
            def _():
                off = wid * _EPW + (b // 16) * _CHUNK
                pltpu.sync_copy(
                    ei_hbm.at[pl.ds((2 * s_id + 1) * _EPAD + off, _CHUNK)],
                    dstage)
                if use_src:
                    pltpu.sync_copy(
                        ei_hbm.at[pl.ds(2 * s_id * _EPAD + off, _CHUNK)],
                        sstage)
            for gg in range(8):
                e0 = (b * 8 + gg) * 16
                ok = e0 + io < _EPW
                d_raw = dstage[pl.ds((b % 16) * 128 + gg * 16, 16)]
                d = jnp.where(ok, d_raw, _DUMP)
                fn(gg, ok, d_raw, d)

        def p1_group(gg, ok, d_raw, d):
            rank, is_last = _rank_and_last(d)
            plsc.addupdate_scatter(hist, [d], rank + 1, mask=is_last)

        @pl.loop(0, _NB_FULL)
        def _(b):
            edge_groups(b, False, p1_group)

        pltpu.sync_copy(hist, shared.at[wid])
        plsc.subcore_barrier()

        @pl.loop(0, _HD // 16)
        def _(i):
            sl = pl.ds(i * 16, 16)
            z = jnp.zeros((16,), jnp.int32)
            prev[sl] = z
            tot[sl] = z

        for w in range(_NSUB):
            pltpu.sync_copy(shared.at[w], row)
            is_prev = jnp.int32(w) < wid

            @pl.loop(0, _HD // 16)
            def _(i, _w=w, _p=is_prev):
                sl = pl.ds(i * 16, 16)
                r = row[sl]
                tot[sl] = tot[sl] + r
                prev[sl] = prev[sl] + jnp.where(_p, r, 0)

        plsc.subcore_barrier()

        def pref_step(i, carry):
            sl = pl.ds(i * 16, 16)
            x = tot[sl]
            incl = plsc.cumsum(x)
            excl = incl - x
            carry = jnp.where(i % 8 == 0, 0, carry)
            base = (i // 8) * EB
            run[sl] = base + carry + excl + prev[sl]
            return carry + jnp.max(incl)

        lax.fori_loop(0, _HD // 16, pref_step, jnp.int32(0))

        @pl.when(wid == 0)
        def _():
            pltpu.sync_copy(tot, cnt_out.at[pl.ds(s_id * _HD, _HD)])

        @pl.loop(0, _NB_FULL)
        def _(b):
            def grp(gg, ok, d_raw, d):
                s = sstage[pl.ds((b % 16) * 128 + gg * 16, 16)]
                rank, is_last = _rank_and_last(d)
                base_v = plsc.load_gather(run, [d])
                slot = s_id * (B + 128) + jnp.where(ok, base_v + rank,
                                                    B + io)
                sidx[pl.ds(gg * 16, 16)] = slot
                sdat[pl.ds(gg * 16, 16)] = s
                ddat[pl.ds(gg * 16, 16)] = d_raw & (NB - 1)
                plsc.addupdate_scatter(run, [d], rank + 1,
                                       mask=is_last & ok)

            edge_groups(b, True, grp)
            pltpu.sync_copy(sdat, src_out.at[sidx])
            pltpu.sync_copy(ddat, dstl_out.at[sidx])

        plsc.subcore_barrier()


def _sc_pack(ei4p):
    @functools.partial(
        pl.kernel,
        out_type=[jax.ShapeDtypeStruct((4 * (B + 128),), jnp.int32),
                  jax.ShapeDtypeStruct((4 * (B + 128),), jnp.int32),
                  jax.ShapeDtypeStruct((4 * _HD,), jnp.int32)],
        mesh=_SC_MESH,
        compiler_params=_SC_CP,
        scratch_types=[
            pltpu.VMEM((_HD,), jnp.int32),
            pltpu.VMEM((_HD,), jnp.int32),
            pltpu.VMEM((_HD,), jnp.int32),
            pltpu.VMEM((_HD,), jnp.int32),
            pltpu.VMEM((_HD,), jnp.int32),
            pltpu.VMEM((_CHUNK,), jnp.int32),
            pltpu.VMEM((_CHUNK,), jnp.int32),
            pltpu.VMEM((128,), jnp.int32),
            pltpu.VMEM((128,), jnp.int32),
            pltpu.VMEM((128,), jnp.int32),
            pltpu.VMEM_SHARED((_NSUB, _HD), jnp.int32),
        ])
    def pk(*args):
        _pack_body(*args)

    return pk(ei4p)


def _all_preps(ei_ll, ei_pp, ei_lp, ei_pl):
    ei4 = jnp.stack([ei_ll, ei_pp, ei_lp, ei_pl])
    ei4p = jnp.pad(ei4, ((0, 0), (0, 0), (0, _EPAD - E))).reshape(-1)
    src4, dstl4, cnt4 = _sc_pack(ei4p)
    src4 = src4.reshape(4, B + 128)
    dstl4 = dstl4.reshape(4, B + 128)
    cnt4 = cnt4.reshape(4, _HD).at[:, _DUMP].set(0)
    blkcnt4 = cnt4.reshape(4, NBLK, NB).sum(-1).astype(jnp.int32)
    preps = []
    for i in range(4):
        dstl = dstl4[i, :B]
        preps.append((src4[i, :B],
                      dstl.reshape(NBLK, 1, EB),
                      dstl.reshape(NBLK, EB, 1),
                      blkcnt4[i].reshape(NBLK, 1, 1)))
    return preps


def _matmul_body(x_ref, w_ref, b_ref, o_ref):
    o_ref[...] = (
        jnp.dot(x_ref[...], w_ref[...], preferred_element_type=jnp.float32, precision=_PREC)
        + b_ref[...])


def _nmatmul(x, w, b):
    k = w.shape[1]
    rows = 1264
    return pl.pallas_call(
        _matmul_body,
        grid=(NPAD // rows,),
        in_specs=[
            pl.BlockSpec((rows, D), lambda i: (i, 0)),
            pl.BlockSpec((D, k), lambda i: (0, 0)),
            pl.BlockSpec((1, k), lambda i: (0, 0)),
        ],
        out_specs=pl.BlockSpec((rows, k), lambda i: (i, 0)),
        out_shape=jax.ShapeDtypeStruct((NPAD, k), jnp.float32),
    )(x, w, b.reshape(1, k))


_SC_MESH = plsc.VectorSubcoreMesh(core_axis_name="c", subcore_axis_name="s")
_SC_CP = pltpu.CompilerParams()
if "needs_layout_passes" in pltpu.CompilerParams.__dataclass_fields__:
    _SC_CP = dataclasses.replace(_SC_CP, needs_layout_passes=False)
_GW = 128


def _gather_rows(table, idx):
    width = table.shape[1]

    @functools.partial(
        pl.kernel,
        out_type=jax.ShapeDtypeStruct((B, width), jnp.float32),
        mesh=_SC_MESH)
    def gk(tab_hbm, idx_hbm, out_hbm):
        def body(i_vmem, o_vmem):
            pltpu.sync_copy(tab_hbm.at[i_vmem.at[0]], o_vmem)

        pltpu.emit_pipeline(
            body,
            grid=(B // _GW,),
            in_specs=[pl.BlockSpec((1, _GW), lambda i: (0, i))],
            out_specs=[pl.BlockSpec((_GW, width), lambda i: (i, 0))],
            core_axis_name=("c", "s"),
            dimension_semantics=(pltpu.PARALLEL,),
        )(idx_hbm, out_hbm)

    return gk(table, idx.reshape(1, B))


def _onehot(dstl_row, valid_row):
    iota_n = lax.broadcasted_iota(jnp.int32, (NB, EB), 0)
    return ((iota_n == dstl_row) & valid_row).astype(jnp.float32)


def _onehot_t(dstl_col, valid_col):
    iota_n = lax.broadcasted_iota(jnp.int32, (EB, NB), 1)
    return ((iota_n == dstl_col) & valid_col).astype(jnp.float32)


def _valids(cnt_ref):
    c = cnt_ref[0, 0, 0]
    iota_r = lax.broadcasted_iota(jnp.int32, (1, EB), 1)
    iota_c = lax.broadcasted_iota(jnp.int32, (EB, 1), 0)
    return iota_r < c, iota_c < c, c


def _softmax_weights(alpha_col, valid_col, m_t, m):
    w = jnp.where(m_t > 0.0, alpha_col, NEG)
    seg_max = jnp.max(w, axis=0, keepdims=True)
    seg_max_col = jnp.reshape(seg_max, (NB, 1))
    m_e = jnp.dot(m_t, seg_max_col,
                  preferred_element_type=jnp.float32, precision=_PREC)
    e_col = jnp.where(valid_col, jnp.exp(alpha_col - m_e), 0.0)
    s = jnp.dot(m, e_col, preferred_element_type=jnp.float32, precision=_PREC)
    return e_col, s


def _trans_body(dstl_r_ref, dstl_c_ref, cnt_ref, q_ref, skip_ref, kvg_ref,
                o_ref):
    dstl_row = dstl_r_ref[0]
    dstl_col = dstl_c_ref[...].reshape(EB, 1)
    valid_row, valid_col, _ = _valids(cnt_ref)
    m = _onehot(dstl_row, valid_row)
    m_t = _onehot_t(dstl_col, valid_col)
    kg = kvg_ref[:, :D]
    vg = kvg_ref[:, D:]
    q_exp = jnp.dot(m_t, q_ref[...], preferred_element_type=jnp.float32, precision=_PREC)
    alpha_col = jnp.sum(q_exp * kg, axis=1, keepdims=True) * SCALE
    e_col, s = _softmax_weights(alpha_col, valid_col, m_t, m)
    acc = jnp.dot(m, e_col * vg, preferred_element_type=jnp.float32, precision=_PREC)
    o_ref[...] = acc / (s + 1e-16) + skip_ref[...]


def _trans_conv(x, prep, wq, bq, wk, bk, wv, bv, ws, bs):
    src_pad, dstl_r, dstl_c, cnt = prep
    wcat = jnp.concatenate([wq, wk, wv, ws], axis=1)
    bcat = jnp.concatenate([bq, bk, bv, bs], axis=0)
    qkvs = _nmatmul(x, wcat, bcat)
    kvg = _gather_rows(qkvs[:, D:3 * D], src_pad)
    return pl.pallas_call(
        _trans_body,
        grid=(NBLK,),
        in_specs=[
            pl.BlockSpec((1, 1, EB), lambda i: (i, 0, 0)),
            pl.BlockSpec((1, EB, 1), lambda i: (i, 0, 0)),
            pl.BlockSpec((1, 1, 1), lambda i: (i, 0, 0)),
            pl.BlockSpec((NB, D), lambda i: (i, 0)),
            pl.BlockSpec((NB, D), lambda i: (i, 0)),
            pl.BlockSpec((EB, 2 * D), lambda i: (i, 0)),
        ],
        out_specs=pl.BlockSpec((NB, D), lambda i: (i, 0)),
        out_shape=jax.ShapeDtypeStruct((NPAD, D), jnp.float32),
    )(dstl_r, dstl_c, cnt, qkvs[:, :D], qkvs[:, 3 * D:], kvg)


def _gat_body(dstl_r_ref, dstl_c_ref, cnt_ref, hd_ref, base_ref, hsg_ref,
              as_ref, ad_ref, b_ref, o_ref):
    dstl_row = dstl_r_ref[0]
    dstl_col = dstl_c_ref[...].reshape(EB, 1)
    valid_row, valid_col, _ = _valids(cnt_ref)
    m = _onehot(dstl_row, valid_row)
    m_t = _onehot_t(dstl_col, valid_col)
    hsg = hsg_ref[...]
    s_src = jnp.sum(hsg * as_ref[...], axis=1, keepdims=True)
    s_dst = jnp.sum(hd_ref[...] * ad_ref[...], axis=1,
                    keepdims=True)
    logits = s_src + jnp.dot(m_t, s_dst,
                             preferred_element_type=jnp.float32, precision=_PREC)
    logits = jnp.where(logits > 0.0, logits, 0.2 * logits)
    e_col, s = _softmax_weights(logits, valid_col, m_t, m)
    acc = jnp.dot(m, e_col * hsg, preferred_element_type=jnp.float32, precision=_PREC)
    o_ref[...] = acc / (s + 1e-16) + b_ref[...] + base_ref[...]


def _gat_conv(x_src, x_dst, prep, w, a_s, a_d, b, base):
    src_pad, dstl_r, dstl_c, cnt = prep
    zero = jnp.zeros((D,), jnp.float32)
    hs = _nmatmul(x_src, w, zero)
    hd = _nmatmul(x_dst, w, zero)
    hsg = _gather_rows(hs, src_pad)
    return pl.pallas_call(
        _gat_body,
        grid=(NBLK,),
        in_specs=[
            pl.BlockSpec((1, 1, EB), lambda i: (i, 0, 0)),
            pl.BlockSpec((1, EB, 1), lambda i: (i, 0, 0)),
            pl.BlockSpec((1, 1, 1), lambda i: (i, 0, 0)),
            pl.BlockSpec((NB, D), lambda i: (i, 0)),
            pl.BlockSpec((NB, D), lambda i: (i, 0)),
            pl.BlockSpec((EB, D), lambda i: (i, 0)),
            pl.BlockSpec((1, D), lambda i: (0, 0)),
            pl.BlockSpec((1, D), lambda i: (0, 0)),
            pl.BlockSpec((1, D), lambda i: (0, 0)),
        ],
        out_specs=pl.BlockSpec((NB, D), lambda i: (i, 0)),
        out_shape=jax.ShapeDtypeStruct((NPAD, D), jnp.float32),
    )(dstl_r, dstl_c, cnt, hd, base, hsg, a_s.reshape(1, D),
      a_d.reshape(1, D), b.reshape(1, D))


_ECHUNK = 1280


def _edge_body(dstl_r_ref, dstl_c_ref, cnt_ref, a_ref, xg_ref, w1b_ref,
               w2_ref, b2_ref, o_ref):
    dstl_row = dstl_r_ref[0]
    dstl_col = dstl_c_ref[...].reshape(EB, 1)
    valid_row, valid_col, c = _valids(cnt_ref)
    iota_c = lax.broadcasted_iota(jnp.int32, (EB, 1), 0)
    m = _onehot(dstl_row, valid_row)
    m_t = _onehot_t(dstl_col, valid_col)

    h2_parts = []
    for c in range(EB // _ECHUNK):
        lo, hi = c * _ECHUNK, (c + 1) * _ECHUNK
        pre = (jnp.dot(m_t[lo:hi], a_ref[...],
                       preferred_element_type=jnp.float32, precision=_PREC)
               + jnp.dot(xg_ref[lo:hi, :], w1b_ref[...],
                         preferred_element_type=jnp.float32, precision=_PREC))
        h = jnp.maximum(pre, 0.0)
        h2_parts.append(
            jnp.dot(h, w2_ref[...], preferred_element_type=jnp.float32, precision=_PREC))
    h2 = jnp.concatenate(h2_parts, axis=0) + b2_ref[...]

    prev = jnp.concatenate(
        [jnp.full((1, 1), -2, jnp.int32), dstl_col[:-1]], axis=0)
    nxt = jnp.concatenate(
        [dstl_col[1:], jnp.full((1, 1), -2, jnp.int32)], axis=0)
    head = jnp.logical_or(dstl_col != prev, jnp.logical_not(valid_col))
    end = jnp.logical_and(valid_col,
                          jnp.logical_or(dstl_col != nxt, iota_c == c - 1))
    v = jnp.where(valid_col, h2, NEG)
    f = head.astype(jnp.float32)
    for d in (1, 2, 4, 8, 16, 32, 64):
        v_sh = jnp.concatenate(
            [jnp.full((d, D), NEG, jnp.float32), v[:-d]], axis=0)
        f_sh = jnp.concatenate(
            [jnp.ones((d, 1), jnp.float32), f[:-d]], axis=0)
        v = jnp.where(f > 0.0, v, jnp.maximum(v, v_sh))
        f = jnp.maximum(f, f_sh)

    picked = jnp.where(end, v, 0.0)
    o_ref[...] = jnp.dot(m, picked, preferred_element_type=jnp.float32, precision=_PREC)


def _edge_conv(x, prep, w1, b1, w2, b2):
    src_pad, dstl_r, dstl_c, cnt = prep
    w1_top, w1_bot = w1[:D], w1[D:]
    a = _nmatmul(x, w1_top - w1_bot, b1)
    xg = _gather_rows(x, src_pad)
    return pl.pallas_call(
        _edge_body,
        grid=(NBLK,),
        in_specs=[
            pl.BlockSpec((1, 1, EB), lambda i: (i, 0, 0)),
            pl.BlockSpec((1, EB, 1), lambda i: (i, 0, 0)),
            pl.BlockSpec((1, 1, 1), lambda i: (i, 0, 0)),
            pl.BlockSpec((NB, 4 * D), lambda i: (i, 0)),
            pl.BlockSpec((EB, D), lambda i: (i, 0)),
            pl.BlockSpec((D, 4 * D), lambda i: (0, 0)),
            pl.BlockSpec((4 * D, D), lambda i: (0, 0)),
            pl.BlockSpec((1, D), lambda i: (0, 0)),
        ],
        out_specs=pl.BlockSpec((NB, D), lambda i: (i, 0)),
        out_shape=jax.ShapeDtypeStruct((NPAD, D), jnp.float32),
    )(dstl_r, dstl_c, cnt, a, xg, w1_bot, w2, b2.reshape(1, D))


def kernel(x_lego, x_point, edge_index_ll, edge_index_pp, edge_index_lp,
           edge_index_pl, trans_Wq, trans_bq, trans_Wk, trans_bk, trans_Wv,
           trans_bv, trans_Ws, trans_bs, edge_W1, edge_b1, edge_W2, edge_b2,
           gatlp_W, gatlp_as, gatlp_ad, gatlp_b, gatpl_W, gatpl_as,
           gatpl_ad, gatpl_b):
    pad = ((0, NPAD - N), (0, 0))
    lego = jnp.pad(x_lego, pad)
    point = jnp.pad(x_point, pad)

    p_ll, p_pp, p_lp, p_pl = _all_preps(
        edge_index_ll, edge_index_pp, edge_index_lp, edge_index_pl)

    for l in range(2):
        sa, sb = 2 * l, 2 * l + 1
        lg = _trans_conv(lego, p_ll, trans_Wq[sa], trans_bq[sa],
                         trans_Wk[sa], trans_bk[sa], trans_Wv[sa],
                         trans_bv[sa], trans_Ws[sa], trans_bs[sa])
        lg = _gat_conv(point, lego, p_pl, gatpl_W[l], gatpl_as[l],
                       gatpl_ad[l], gatpl_b[l], lg)
        pt = _edge_conv(point, p_pp, edge_W1[sa], edge_b1[sa],
                        edge_W2[sa], edge_b2[sa])
        pt = _gat_conv(lego, point, p_lp, gatlp_W[l], gatlp_as[l],
                       gatlp_ad[l], gatlp_b[l], pt)
        lego = _trans_conv(lg, p_ll, trans_Wq[sb], trans_bq[sb],
                           trans_Wk[sb], trans_bk[sb], trans_Wv[sb],
                           trans_bv[sb], trans_Ws[sb], trans_bs[sb])
        point = _edge_conv(pt, p_pp, edge_W1[sb], edge_b1[sb],
                           edge_W2[sb], edge_b2[sb])

    return lego[:N], point[:N]
